# Initial kernel scaffold; baseline (speedup 1.0000x reference)
import jax
import jax.numpy as jnp
from jax.experimental import pallas as pl


def kernel(input, card, rank, suit):
    raise NotImplementedError("write your pallas kernel here")


# speedup vs baseline: 11.0048x; 11.0048x over previous
import functools

import jax
import jax.numpy as jnp
from jax import lax
from jax.experimental import pallas as pl
from jax.experimental.pallas import tpu as pltpu
from jax.experimental.pallas import tpu_sc as plsc

DIM = 256
B = 16384
NUM_CARDS = 7
NUM_ROWS = 52

NC = 2
NS = 16
NW = NC * NS
L = 16

B_PER_W = B // NW
OUT_TILE = 128
N_TILES = B_PER_W // OUT_TILE
DJ = DIM // L


def _splat(ref, addr):
    return plsc.load_gather(ref, [jnp.broadcast_to(addr, (L,))])


def _sc_lookup(idx_hbm, card_hbm, rank_hbm, suit_hbm, out_hbm,
               idx_v, tab_v, rank_v, suit_v, out_v):
    wid = lax.axis_index("s") * NC + lax.axis_index("c")
    base = wid * B_PER_W

    pltpu.sync_copy(idx_hbm.at[pl.ds(base * NUM_CARDS, B_PER_W * NUM_CARDS)],
                    idx_v)
    pltpu.sync_copy(card_hbm, tab_v)
    pltpu.sync_copy(rank_hbm, rank_v)
    pltpu.sync_copy(suit_hbm, suit_v)

    cols = [jax.lax.broadcasted_iota(jnp.int32, (L,), 0) + L * j
            for j in range(DJ)]

    def build_row(i, _):
        rb = (i >> 2) * DIM
        sb = (i & 3) * DIM
        tb = i * DIM
        for j in range(DJ):
            t = plsc.load_gather(tab_v, [tb + cols[j]])
            r = plsc.load_gather(rank_v, [rb + cols[j]])
            s = plsc.load_gather(suit_v, [sb + cols[j]])
            out = t + r + s
            plsc.store_scatter(tab_v, [tb + cols[j]], out)
        return 0

    lax.fori_loop(0, NUM_ROWS, build_row, 0, unroll=False)

    def do_row(b, k):
        row_bases = [_splat(idx_v, (k * OUT_TILE + b) * NUM_CARDS + c) * DIM
                     for c in range(NUM_CARDS)]
        for j in range(DJ):
            acc = plsc.load_gather(tab_v, [row_bases[0] + cols[j]])
            for c in range(1, NUM_CARDS):
                acc = acc + plsc.load_gather(tab_v, [row_bases[c] + cols[j]])
            out_v[b, pl.ds(L * j, L)] = acc
        return k

    for k in range(N_TILES):
        lax.fori_loop(0, OUT_TILE, do_row, k, unroll=False)
        pltpu.sync_copy(out_v, out_hbm.at[pl.ds(base + k * OUT_TILE, OUT_TILE)])


@jax.jit
def kernel(input, card, rank, suit):
    idx = input.astype(jnp.int32).reshape(-1)
    mesh = plsc.VectorSubcoreMesh(core_axis_name="c", subcore_axis_name="s",
                                  num_cores=NC, num_subcores=NS)
    call = pl.kernel(
        _sc_lookup,
        out_type=jax.ShapeDtypeStruct((B, DIM), jnp.float32),
        mesh=mesh,
        scratch_types=[
            pltpu.VMEM((B_PER_W * NUM_CARDS,), jnp.int32),
            pltpu.VMEM((NUM_ROWS * DIM,), jnp.float32),
            pltpu.VMEM((13 * DIM,), jnp.float32),
            pltpu.VMEM((4 * DIM,), jnp.float32),
            pltpu.VMEM((OUT_TILE, DIM), jnp.float32),
        ],
        compiler_params=pltpu.CompilerParams(
            needs_layout_passes=False,
            use_tc_tiling_on_sc=False,
        ),
    )
    return call(idx, card.reshape(-1), rank.reshape(-1), suit.reshape(-1))


# speedup vs baseline: 26.7332x; 2.4292x over previous
import jax
import jax.numpy as jnp
from jax import lax
from jax.experimental import pallas as pl
from jax.experimental.pallas import tpu as pltpu
from jax.experimental.pallas import tpu_sc as plsc

DIM = 256
B = 16384
NUM_CARDS = 7
NUM_ROWS = 52
KPAD = 64

NC = 2
NS = 16
NW = NC * NS
L = 16

B_PER_W = B // NW
GROUPS = B_PER_W // L
DJ = DIM // L


def _sc_hist(idx_hbm, card_hbm, rank_hbm, suit_hbm, cnt_hbm, tab_hbm,
             idx_v, cnt_v, tab_v, rank_v, suit_v):
    wid = lax.axis_index("s") * NC + lax.axis_index("c")
    base = wid * B_PER_W

    pltpu.sync_copy(idx_hbm.at[pl.ds(base * NUM_CARDS, B_PER_W * NUM_CARDS)],
                    idx_v)

    iot = jax.lax.broadcasted_iota(jnp.int32, (L,), 0)
    zeros = jnp.zeros((L,), jnp.float32)
    ones = jnp.ones((L,), jnp.float32)

    def zero_body(i, _):
        for u in range(8):
            cnt_v[pl.ds((i * 8 + u) * L, L)] = zeros
        return 0
    lax.fori_loop(0, B_PER_W * KPAD // (8 * L), zero_body, 0, unroll=False)

    iot7 = iot * NUM_CARDS
    iot64 = iot * KPAD
    def hist_body(g, _):
        a7 = g * (L * NUM_CARDS) + iot7
        bbase = g * (L * KPAD) + iot64
        for c in range(NUM_CARDS):
            v = plsc.load_gather(idx_v, [a7 + c])
            plsc.addupdate_scatter(cnt_v, [bbase + v], ones)
        return 0
    lax.fori_loop(0, GROUPS, hist_body, 0, unroll=False)

    pltpu.sync_copy(cnt_v, cnt_hbm.at[pl.ds(base * KPAD, B_PER_W * KPAD)])

    @pl.when(wid == 0)
    def _():
        pltpu.sync_copy(card_hbm, tab_v.at[pl.ds(0, NUM_ROWS * DIM)])
        pltpu.sync_copy(rank_hbm, rank_v)
        pltpu.sync_copy(suit_hbm, suit_v)
        cols = [iot + L * j for j in range(DJ)]

        def build_row(i, _):
            rb = (i >> 2) * DIM
            sb = (i & 3) * DIM
            tb = i * DIM
            for j in range(DJ):
                t = plsc.load_gather(tab_v, [tb + cols[j]])
                r = plsc.load_gather(rank_v, [rb + cols[j]])
                s = plsc.load_gather(suit_v, [sb + cols[j]])
                plsc.store_scatter(tab_v, [tb + cols[j]], t + r + s)
            return 0
        lax.fori_loop(0, NUM_ROWS, build_row, 0, unroll=False)

        def pad_row(i, _):
            for j in range(DJ):
                tab_v[pl.ds(i * DIM + L * j, L)] = zeros
            return 0
        lax.fori_loop(NUM_ROWS, KPAD, pad_row, 0, unroll=False)
        pltpu.sync_copy(tab_v, tab_hbm)


def _tc_matmul(cnt_ref, tab_ref, out_ref):
    out_ref[...] = jnp.dot(cnt_ref[...], tab_ref[...],
                           preferred_element_type=jnp.float32)


BLK = 2048


@jax.jit
def kernel(input, card, rank, suit):
    idx = input.astype(jnp.int32).reshape(-1)
    mesh = plsc.VectorSubcoreMesh(core_axis_name="c", subcore_axis_name="s",
                                  num_cores=NC, num_subcores=NS)
    sc_call = pl.kernel(
        _sc_hist,
        out_type=(
            jax.ShapeDtypeStruct((B * KPAD,), jnp.float32),
            jax.ShapeDtypeStruct((KPAD * DIM,), jnp.float32),
        ),
        mesh=mesh,
        scratch_types=[
            pltpu.VMEM((B_PER_W * NUM_CARDS,), jnp.int32),
            pltpu.VMEM((B_PER_W * KPAD,), jnp.float32),
            pltpu.VMEM((KPAD * DIM,), jnp.float32),
            pltpu.VMEM((13 * DIM,), jnp.float32),
            pltpu.VMEM((4 * DIM,), jnp.float32),
        ],
        compiler_params=pltpu.CompilerParams(
            needs_layout_passes=False,
            use_tc_tiling_on_sc=False,
        ),
    )
    counts, tab = sc_call(idx, card.reshape(-1), rank.reshape(-1),
                          suit.reshape(-1))
    counts = counts.reshape(B, KPAD)
    tab = tab.reshape(KPAD, DIM)
    return pl.pallas_call(
        _tc_matmul,
        grid=(B // BLK,),
        in_specs=[
            pl.BlockSpec((BLK, KPAD), lambda i: (i, 0)),
            pl.BlockSpec((KPAD, DIM), lambda i: (0, 0)),
        ],
        out_specs=pl.BlockSpec((BLK, DIM), lambda i: (i, 0)),
        out_shape=jax.ShapeDtypeStruct((B, DIM), jnp.float32),
    )(counts, tab)
